# Initial kernel scaffold; baseline (speedup 1.0000x reference)
import jax
import jax.numpy as jnp
from jax.experimental import pallas as pl


def kernel(x, edge_index, old_index, W_gcn, b_gcn, weight):
    raise NotImplementedError("write your pallas kernel here")


# speedup vs baseline: 14.0731x; 14.0731x over previous
import functools

import jax
import jax.numpy as jnp
from jax import lax
from jax.experimental import pallas as pl
from jax.experimental.pallas import tpu as pltpu
from jax.experimental.pallas import tpu_sc as plsc

N = 10000
C = 128
HEADS = 6
E = 320000
TRASH = N
NPAD = 10240
SELF_PAD = 10240
EP = 331776
CHUNK = 128
CHUNKS_PER_TILE = 81
EDGES_PER_TILE = CHUNKS_PER_TILE * CHUNK
ROWS_PER_TILE = NPAD // 16

_mesh = plsc.VectorSubcoreMesh(core_axis_name="c", subcore_axis_name="s")


_EROWS = EP // CHUNK


def _adj_body(row_ref, col_ref, adj_ref):
    r = row_ref[...]
    c = col_ref[...]
    rowid = lax.broadcasted_iota(jnp.int32, (_EROWS, CHUNK), 0)
    in_real = rowid >= (SELF_PAD // CHUNK)
    adj_ref[...] = jnp.where((r == c) & in_real, TRASH, c)


def _adj_call(row2d, col2d):
    return pl.pallas_call(
        _adj_body,
        out_shape=jax.ShapeDtypeStruct((_EROWS, CHUNK), jnp.int32),
    )(row2d, col2d)


def _deg_body(adj_hbm, deg_out, adj_v, ones_v, zrow_v, acc):
    cid = lax.axis_index("c")
    sid = lax.axis_index("s")
    wid = cid * 16 + sid

    ones16 = jnp.ones((16,), jnp.float32)
    zeros16 = jnp.zeros((16,), jnp.float32)
    for j in range(CHUNK // 16):
        ones_v[pl.ds(j * 16, 16)] = ones16
    for j in range(ROWS_PER_TILE // 16):
        zrow_v[pl.ds(j * 16, 16)] = zeros16
    pltpu.sync_copy(zrow_v, acc.at[pl.ds(sid * ROWS_PER_TILE, ROWS_PER_TILE)])
    plsc.subcore_barrier()

    base0 = wid * EDGES_PER_TILE

    def chunk_body(i, carry):
        base = base0 + i * CHUNK
        pltpu.sync_copy(adj_hbm.at[pl.ds(base, CHUNK)], adj_v)
        pltpu.sync_copy(ones_v, acc.at[adj_v], add=True)
        return carry

    lax.fori_loop(0, CHUNKS_PER_TILE, chunk_body, 0)
    plsc.subcore_barrier()
    pltpu.sync_copy(acc.at[pl.ds(sid * ROWS_PER_TILE, ROWS_PER_TILE)],
                    deg_out.at[cid, pl.ds(sid * ROWS_PER_TILE, ROWS_PER_TILE)])


_deg_call = functools.partial(
    pl.kernel,
    out_type=jax.ShapeDtypeStruct((2, NPAD), jnp.float32),
    mesh=_mesh,
    scratch_types=[
        pltpu.VMEM((CHUNK,), jnp.int32),
        pltpu.VMEM((CHUNK,), jnp.float32),
        pltpu.VMEM((ROWS_PER_TILE,), jnp.float32),
        pltpu.VMEM_SHARED((NPAD,), jnp.float32),
    ],
)(_deg_body)


def _agg_body(g_hbm, row_hbm, adj_hbm, out_hbm,
              row_v, adj_v, rows_v, zblk, acc, sem):
    cid = lax.axis_index("c")
    sid = lax.axis_index("s")
    wid = cid * 16 + sid

    zeros16 = jnp.zeros((16,), jnp.float32)

    def zrow(i, carry):
        for j in range(C // 16):
            zblk[i, pl.ds(j * 16, 16)] = zeros16
        return carry

    lax.fori_loop(0, 64, zrow, 0)

    def zacc(k, carry):
        pltpu.sync_copy(zblk, acc.at[pl.ds(sid * ROWS_PER_TILE + k * 64, 64)])
        return carry

    lax.fori_loop(0, ROWS_PER_TILE // 64, zacc, 0)
    plsc.subcore_barrier()

    base0 = wid * EDGES_PER_TILE

    def chunk_body(i, carry):
        base = base0 + i * CHUNK
        pltpu.sync_copy(row_hbm.at[pl.ds(base, CHUNK)], row_v)
        pltpu.sync_copy(adj_hbm.at[pl.ds(base, CHUNK)], adj_v)
        pltpu.async_copy(g_hbm.at[row_v], rows_v, sem).wait()
        pltpu.sync_copy(rows_v, acc.at[adj_v], add=True)
        return carry

    lax.fori_loop(0, CHUNKS_PER_TILE, chunk_body, 0)
    plsc.subcore_barrier()
    pltpu.sync_copy(acc.at[pl.ds(sid * ROWS_PER_TILE, ROWS_PER_TILE)],
                    out_hbm.at[cid, pl.ds(sid * ROWS_PER_TILE, ROWS_PER_TILE)])


_agg_call = functools.partial(
    pl.kernel,
    out_type=jax.ShapeDtypeStruct((2, NPAD, C), jnp.float32),
    mesh=_mesh,
    scratch_types=[
        pltpu.VMEM((CHUNK,), jnp.int32),
        pltpu.VMEM((CHUNK,), jnp.int32),
        pltpu.VMEM((CHUNK, C), jnp.float32),
        pltpu.VMEM((64, C), jnp.float32),
        pltpu.VMEM_SHARED((NPAD, C), jnp.float32),
        pltpu.SemaphoreType.DMA,
    ],
)(_agg_body)


_RB = 640


def _g_body(deg_ref, x_ref, w_ref, g_ref):
    p = deg_ref[...]
    s = lax.rsqrt(p[0] + p[1])
    h = jnp.dot(x_ref[...], w_ref[...], preferred_element_type=jnp.float32)
    g_ref[...] = s[:, None] * h


def _g_call(deg_parts, x, W_gcn):
    return pl.pallas_call(
        _g_body,
        grid=(NPAD // _RB,),
        in_specs=[
            pl.BlockSpec((2, _RB), lambda i: (0, i)),
            pl.BlockSpec((_RB, C), lambda i: (i, 0)),
            pl.BlockSpec((C, C), lambda i: (0, 0)),
        ],
        out_specs=pl.BlockSpec((_RB, C), lambda i: (i, 0)),
        out_shape=jax.ShapeDtypeStruct((N, C), jnp.float32),
    )(deg_parts, x, W_gcn)


def _out_body(deg_ref, a_ref, b_ref, w_ref, o_ref):
    p = deg_ref[...]
    a = a_ref[...]
    s = lax.rsqrt(p[0] + p[1])
    out1 = s[:, None] * (a[0] + a[1]) + b_ref[...]
    o_ref[...] = jnp.dot(out1, w_ref[...], preferred_element_type=jnp.float32)


def _out_call(deg_parts, a_parts, b2d, weight):
    return pl.pallas_call(
        _out_body,
        grid=(NPAD // _RB,),
        in_specs=[
            pl.BlockSpec((2, _RB), lambda i: (0, i)),
            pl.BlockSpec((2, _RB, C), lambda i: (0, i, 0)),
            pl.BlockSpec((1, C), lambda i: (0, 0)),
            pl.BlockSpec((C, HEADS * C), lambda i: (0, 0)),
        ],
        out_specs=pl.BlockSpec((_RB, HEADS * C), lambda i: (i, 0)),
        out_shape=jax.ShapeDtypeStruct((N, HEADS * C), jnp.float32),
    )(deg_parts, a_parts, b2d, weight)


@jax.jit
def kernel(x, edge_index, old_index, W_gcn, b_gcn, weight):
    row0, col0 = edge_index[0], edge_index[1]
    loop = jnp.arange(N, dtype=jnp.int32)
    row_full = jnp.concatenate([
        loop, jnp.zeros((SELF_PAD - N,), jnp.int32),
        row0, jnp.zeros((EP - SELF_PAD - E,), jnp.int32)])
    col_full = jnp.concatenate([
        loop, jnp.full((SELF_PAD - N,), TRASH, jnp.int32),
        col0, jnp.full((EP - SELF_PAD - E,), TRASH, jnp.int32)])

    col_adj = _adj_call(row_full.reshape(_EROWS, CHUNK),
                        col_full.reshape(_EROWS, CHUNK)).reshape(EP)
    deg_parts = _deg_call(col_adj)
    g = _g_call(deg_parts, x, W_gcn)
    a_parts = _agg_call(g, row_full, col_adj)
    out = _out_call(deg_parts, a_parts, b_gcn.reshape(1, C), weight)
    return out.reshape(N, HEADS, C)
